# baseline (speedup 1.0000x reference)
import functools

import jax
import jax.numpy as jnp
from jax import lax
from jax.experimental import pallas as pl
from jax.experimental.pallas import tpu as pltpu
from jax.experimental.pallas import tpu_sc as plsc

VOCAB = 1000000
HALF = VOCAB // 2
EMB = 64
B = 16384
L = 50

NUM_ROWS = B * L
NW = 32
PADL = 56

_MESH = dict(mesh=plsc.VectorSubcoreMesh(core_axis_name="c", subcore_axis_name="s"))


def _wid():
    return lax.axis_index("s") * 2 + lax.axis_index("c")


TAB_CHUNK = 160
TAB_CHUNKS = HALF // TAB_CHUNK
TAB_K = (TAB_CHUNKS + NW - 1) // NW


@functools.partial(
    pl.kernel,
    out_type=jax.ShapeDtypeStruct((HALF, 2 * EMB), jnp.float32),
    scratch_types=[
        pltpu.VMEM((2, 2 * TAB_CHUNK, EMB), jnp.float32),
        pltpu.VMEM((2, TAB_CHUNK, 2 * EMB), jnp.float32),
        pltpu.SemaphoreType.DMA((2,)),
    ],
    **_MESH,
)
def _table_repack(table_hbm, wide_hbm, ibuf, obuf, isem):
    wid = _wid()

    def fire(k, slot):
        c = wid + NW * k

        @pl.when(c < TAB_CHUNKS)
        def _():
            pltpu.async_copy(
                table_hbm.at[pl.ds(c * 2 * TAB_CHUNK, 2 * TAB_CHUNK)],
                ibuf.at[slot], isem.at[slot],
            )

    def wait_in(slot):
        pltpu.make_async_copy(
            table_hbm.at[pl.ds(0, 2 * TAB_CHUNK)], ibuf.at[slot], isem.at[slot]
        ).wait()

    fire(0, 0)
    fire(1, 1)

    def kbody(kk, carry):
        for s in range(2):
            k = kk * 2 + s
            c = wid + NW * k

            @pl.when(c < TAB_CHUNKS)
            def _():
                wait_in(s)

                def repack(jb, carry2):
                    for jj in range(4):
                        j = jb * 4 + jj
                        for h in range(2):
                            for g in range(4):
                                lo = h * EMB + g * 16
                                obuf[s, j, lo:lo + 16] = (
                                    ibuf[s, 2 * j + h, g * 16:(g + 1) * 16]
                                )
                    return carry2

                lax.fori_loop(0, TAB_CHUNK // 4, repack, 0)
                pltpu.sync_copy(
                    obuf.at[s], wide_hbm.at[pl.ds(c * TAB_CHUNK, TAB_CHUNK)]
                )

            fire(k + 2, s)
        return carry

    lax.fori_loop(0, (TAB_K + 1) // 2, kbody, 0)


IDX_MINOR = 100
IDX_ROWS = NUM_ROWS // IDX_MINOR
ROWS_PER_W = NUM_ROWS // NW
BLOCKS_PER_W = IDX_ROWS // NW
STREAMS = 4
CHUNK = STREAMS * IDX_MINOR
BATCH_PER_CHUNK = CHUNK // L
NBUF = 2
CHUNKS = BLOCKS_PER_W // STREAMS
GROUPS = CHUNKS // NBUF


@functools.partial(
    pl.kernel,
    out_type=jax.ShapeDtypeStruct((NUM_ROWS, EMB), jnp.float32),
    scratch_types=[
        pltpu.VMEM((BLOCKS_PER_W, IDX_MINOR), jnp.int32),
        pltpu.VMEM((NBUF, CHUNK, EMB), jnp.float32),
        pltpu.SemaphoreType.DMA((NBUF,)),
        pltpu.SemaphoreType.DMA((NBUF,)),
    ],
    compiler_params=pltpu.CompilerParams(use_tc_tiling_on_sc=False),
    **_MESH,
)
def _gather_kernel(table_hbm, idx_hbm, out_hbm, idx_v, rows_v, gsem, ssem):
    wid = _wid()
    base_blk = wid * BLOCKS_PER_W
    base_b = wid * (ROWS_PER_W // L)

    pltpu.sync_copy(idx_hbm.at[pl.ds(base_blk, BLOCKS_PER_W)], idx_v)

    def fire_gather(chunk, slot):
        for j in range(STREAMS):
            pltpu.async_copy(
                table_hbm.at[idx_v.at[chunk * STREAMS + j]],
                rows_v.at[slot].at[pl.ds(j * IDX_MINOR, IDX_MINOR)],
                gsem.at[slot],
            )

    def wait_gather(slot):
        pltpu.make_async_copy(
            table_hbm.at[pl.ds(0, CHUNK)], rows_v.at[slot], gsem.at[slot]
        ).wait()

    def dummy_store(slot):
        return pltpu.make_async_copy(
            rows_v.at[slot], out_hbm.at[pl.ds(0, CHUNK)], ssem.at[slot]
        )

    for s in range(NBUF):
        fire_gather(s, s)

    def group_body(g, carry):
        for s in range(NBUF):
            i = g * NBUF + s
            wait_gather(s)
            pltpu.async_copy(
                rows_v.at[s],
                out_hbm.at[pl.ds((base_blk + i * STREAMS) * IDX_MINOR, CHUNK)],
                ssem.at[s],
            )
            dummy_store(s).wait()

            @pl.when(g < GROUPS - 1)
            def _():
                fire_gather(i + NBUF, s)

        return carry

    lax.fori_loop(0, GROUPS, group_body, 0)


def kernel(data, ivectors):
    wide = _table_repack(ivectors)
    view = wide.reshape(VOCAB, EMB)
    idx = data.reshape(-1).astype(jnp.int32).reshape(IDX_ROWS, IDX_MINOR)
    out = _gather_kernel(view, idx)
    return out.reshape(B, L, EMB)


# speedup vs baseline: 1.1884x; 1.1884x over previous
import functools

import jax
import jax.numpy as jnp
from jax import lax
from jax.experimental import pallas as pl
from jax.experimental.pallas import tpu as pltpu
from jax.experimental.pallas import tpu_sc as plsc

VOCAB = 1000000
HALF = VOCAB // 2
EMB = 64
B = 16384
L = 50

NUM_ROWS = B * L
NW = 32
PADL = 56

_MESH = dict(mesh=plsc.VectorSubcoreMesh(core_axis_name="c", subcore_axis_name="s"))


def _wid():
    return lax.axis_index("s") * 2 + lax.axis_index("c")


TAB_CHUNK = 160
TAB_CHUNKS = HALF // TAB_CHUNK
TAB_K = (TAB_CHUNKS + NW - 1) // NW


@functools.partial(
    pl.kernel,
    out_type=jax.ShapeDtypeStruct((HALF, 2 * EMB), jnp.float32),
    scratch_types=[
        pltpu.VMEM((2, 2 * TAB_CHUNK, EMB), jnp.float32),
        pltpu.VMEM((2, TAB_CHUNK, 2 * EMB), jnp.float32),
        pltpu.SemaphoreType.DMA((2,)),
    ],
    **_MESH,
)
def _table_repack(table_hbm, wide_hbm, ibuf, obuf, isem):
    wid = _wid()

    def fire(k, slot):
        c = wid + NW * k

        @pl.when(c < TAB_CHUNKS)
        def _():
            pltpu.async_copy(
                table_hbm.at[pl.ds(c * 2 * TAB_CHUNK, 2 * TAB_CHUNK)],
                ibuf.at[slot], isem.at[slot],
            )

    def wait_in(slot):
        pltpu.make_async_copy(
            table_hbm.at[pl.ds(0, 2 * TAB_CHUNK)], ibuf.at[slot], isem.at[slot]
        ).wait()

    fire(0, 0)
    fire(1, 1)

    def kbody(kk, carry):
        for s in range(2):
            k = kk * 2 + s
            c = wid + NW * k

            @pl.when(c < TAB_CHUNKS)
            def _():
                wait_in(s)

                @plsc.parallel_loop(0, TAB_CHUNK, unroll=8)
                def repack(j):
                    for h in range(2):
                        for g in range(4):
                            lo = h * EMB + g * 16
                            obuf[s, j, lo:lo + 16] = (
                                ibuf[s, 2 * j + h, g * 16:(g + 1) * 16]
                            )
                pltpu.sync_copy(
                    obuf.at[s], wide_hbm.at[pl.ds(c * TAB_CHUNK, TAB_CHUNK)]
                )

            fire(k + 2, s)
        return carry

    lax.fori_loop(0, (TAB_K + 1) // 2, kbody, 0)


IDX_MINOR = 100
IDX_ROWS = NUM_ROWS // IDX_MINOR
ROWS_PER_W = NUM_ROWS // NW
BLOCKS_PER_W = IDX_ROWS // NW
STREAMS = 4
CHUNK = STREAMS * IDX_MINOR
BATCH_PER_CHUNK = CHUNK // L
NBUF = 2
CHUNKS = BLOCKS_PER_W // STREAMS
GROUPS = CHUNKS // NBUF


@functools.partial(
    pl.kernel,
    out_type=jax.ShapeDtypeStruct((NUM_ROWS, EMB), jnp.float32),
    scratch_types=[
        pltpu.VMEM((BLOCKS_PER_W, IDX_MINOR), jnp.int32),
        pltpu.VMEM((NBUF, CHUNK, EMB), jnp.float32),
        pltpu.SemaphoreType.DMA((NBUF,)),
        pltpu.SemaphoreType.DMA((NBUF,)),
    ],
    compiler_params=pltpu.CompilerParams(use_tc_tiling_on_sc=False),
    **_MESH,
)
def _gather_kernel(table_hbm, idx_hbm, out_hbm, idx_v, rows_v, gsem, ssem):
    wid = _wid()
    base_blk = wid * BLOCKS_PER_W
    base_b = wid * (ROWS_PER_W // L)

    pltpu.sync_copy(idx_hbm.at[pl.ds(base_blk, BLOCKS_PER_W)], idx_v)

    def fire_gather(chunk, slot):
        for j in range(STREAMS):
            pltpu.async_copy(
                table_hbm.at[idx_v.at[chunk * STREAMS + j]],
                rows_v.at[slot].at[pl.ds(j * IDX_MINOR, IDX_MINOR)],
                gsem.at[slot],
            )

    def wait_gather(slot):
        pltpu.make_async_copy(
            table_hbm.at[pl.ds(0, CHUNK)], rows_v.at[slot], gsem.at[slot]
        ).wait()

    def dummy_store(slot):
        return pltpu.make_async_copy(
            rows_v.at[slot], out_hbm.at[pl.ds(0, CHUNK)], ssem.at[slot]
        )

    for s in range(NBUF):
        fire_gather(s, s)

    def group_body(g, carry):
        for s in range(NBUF):
            i = g * NBUF + s
            wait_gather(s)
            pltpu.async_copy(
                rows_v.at[s],
                out_hbm.at[pl.ds((base_blk + i * STREAMS) * IDX_MINOR, CHUNK)],
                ssem.at[s],
            )
            dummy_store(s).wait()

            @pl.when(g < GROUPS - 1)
            def _():
                fire_gather(i + NBUF, s)

        return carry

    lax.fori_loop(0, GROUPS, group_body, 0)


def kernel(data, ivectors):
    wide = _table_repack(ivectors)
    view = wide.reshape(VOCAB, EMB)
    idx = data.reshape(-1).astype(jnp.int32).reshape(IDX_ROWS, IDX_MINOR)
    out = _gather_kernel(view, idx)
    return out.reshape(B, L, EMB)


# speedup vs baseline: 1.6142x; 1.3583x over previous
import functools

import jax
import jax.numpy as jnp
from jax import lax
from jax.experimental import pallas as pl
from jax.experimental.pallas import tpu as pltpu
from jax.experimental.pallas import tpu_sc as plsc

VOCAB = 1000000
HALF = VOCAB // 2
EMB = 64
B = 16384
L = 50

NUM_ROWS = B * L
NW = 32
PADL = 56

_MESH = dict(mesh=plsc.VectorSubcoreMesh(core_axis_name="c", subcore_axis_name="s"))


def _wid():
    return lax.axis_index("s") * 2 + lax.axis_index("c")


IDX_MINOR = 100
IDX_ROWS = NUM_ROWS // IDX_MINOR
ROWS_PER_W = NUM_ROWS // NW
BLOCKS_PER_W = IDX_ROWS // NW
STREAMS = 4
CHUNK = STREAMS * IDX_MINOR
BATCH_PER_CHUNK = CHUNK // L
NBUF = 2
CHUNKS = BLOCKS_PER_W // STREAMS
GROUPS = CHUNKS // NBUF


@functools.partial(
    pl.kernel,
    out_type=jax.ShapeDtypeStruct((B, PADL, 2 * EMB), jnp.float32),
    scratch_types=[
        pltpu.VMEM((BLOCKS_PER_W, IDX_MINOR), jnp.int32),
        pltpu.VMEM((NBUF, CHUNK, EMB), jnp.float32),
        pltpu.SemaphoreType.DMA((NBUF,)),
        pltpu.SemaphoreType.DMA((NBUF,)),
    ],
    compiler_params=pltpu.CompilerParams(use_tc_tiling_on_sc=False),
    **_MESH,
)
def _gather_kernel(table_hbm, idx_hbm, out_hbm, idx_v, rows_v, gsem, ssem):
    wid = _wid()
    base_blk = wid * BLOCKS_PER_W
    base_b = wid * (ROWS_PER_W // L)

    pltpu.sync_copy(idx_hbm.at[pl.ds(base_blk, BLOCKS_PER_W)], idx_v)

    def fire_gather(chunk, slot):
        for j in range(STREAMS):
            pltpu.async_copy(
                table_hbm.at[idx_v.at[chunk * STREAMS + j]],
                rows_v.at[slot].at[pl.ds(j * IDX_MINOR, IDX_MINOR)],
                gsem.at[slot],
            )

    def wait_gather(slot):
        pltpu.make_async_copy(
            table_hbm.at[pl.ds(0, CHUNK)], rows_v.at[slot], gsem.at[slot]
        ).wait()

    def dummy_store(slot):
        return pltpu.make_async_copy(
            rows_v.at[slot].at[pl.ds(0, L)],
            out_hbm.at[0].at[pl.ds(0, L), pl.ds(0, EMB)],
            ssem.at[slot],
        )

    for s in range(NBUF):
        fire_gather(s, s)

    def group_body(g, carry):
        for s in range(NBUF):
            i = g * NBUF + s
            wait_gather(s)
            for j in range(BATCH_PER_CHUNK):
                b = base_b + i * BATCH_PER_CHUNK + j
                pltpu.async_copy(
                    rows_v.at[s].at[pl.ds(j * L, L)],
                    out_hbm.at[b].at[pl.ds(0, L), pl.ds(0, EMB)],
                    ssem.at[s],
                )
            for j in range(BATCH_PER_CHUNK):
                dummy_store(s).wait()

            @pl.when(g < GROUPS - 1)
            def _():
                fire_gather(i + NBUF, s)

        return carry

    lax.fori_loop(0, GROUPS, group_body, 0)


def kernel(data, ivectors):
    idx = data.reshape(-1).astype(jnp.int32).reshape(IDX_ROWS, IDX_MINOR)
    big = _gather_kernel(ivectors, idx)
    return big[:, :L, :EMB]
